# Initial kernel scaffold; baseline (speedup 1.0000x reference)
import jax
import jax.numpy as jnp
from jax.experimental import pallas as pl


def kernel(feat_user, feat_item, ei_follows, ei_clicks, ei_clicked_by, W_follows, b_follows, W_clicks, b_clicks, W_clicked_by, b_clicked_by):
    raise NotImplementedError("write your pallas kernel here")


# speedup vs baseline: 3.8705x; 3.8705x over previous
import dataclasses

import jax
import jax.numpy as jnp
from jax.experimental import pallas as pl
from jax.experimental.pallas import tpu as pltpu
from jax.experimental.pallas import tpu_sc as plsc

N = 10000
E = 160000
D = 256
HALF = 128
NSUB = 16
EPW = E // NSUB
CHUNK = 200
NCHUNK = EPW // CHUNK
NPAD = 10240
ROWS_PER_SUB = NPAD // NSUB
ZROWS = 32
EPWH = EPW // 2


def _sc_body(au, bu, ai, bi, sf, df, sc_, dc, scb, dcb,
             o_fa, o_fb, o_ca, o_cb_, o_ba, o_bb, o_cf, o_cc0, o_cc1, o_cb2,
             acc, rows, sidx, didx, zbuf):
    cid = jax.lax.axis_index("c")
    sid = jax.lax.axis_index("s")

    @pl.loop(0, ZROWS)
    def _(r):
        @pl.loop(0, HALF, step=16)
        def _(c):
            zbuf[r, pl.ds(c, 16)] = jnp.zeros((16,), jnp.float32)

    tasks = (
        (au, sf, df, o_fa, 0, False),
        (bu, sf, df, o_fb, 1, False),
        (au, sc_, dc, o_ca, 1, False),
        (bu, sc_, dc, o_cb_, 0, False),
        (ai, scb, dcb, o_ba, 0, False),
        (bi, scb, dcb, o_bb, 1, False),
        (None, None, df, o_cf, 0, False),
        (None, None, dc, o_cc0, 0, True),
        (None, None, dc, o_cc1, 1, True),
        (None, None, dcb, o_cb2, 1, False),
    )

    for feat, src, dst, out, core, half in tasks:
        count = feat is None
        epw_t = EPWH if half else EPW
        nchunk_t = epw_t // CHUNK
        off = core * NSUB * EPWH if half else 0

        @pl.when(cid == core)
        def _():
            @pl.loop(0, ROWS_PER_SUB, step=ZROWS)
            def _(r):
                pltpu.sync_copy(zbuf, acc.at[pl.ds(sid * ROWS_PER_SUB + r, ZROWS)])
            plsc.subcore_barrier()

            if count:
                @pl.loop(0, CHUNK)
                def _(r):
                    @pl.loop(0, HALF, step=16)
                    def _(c):
                        rows[r, pl.ds(c, 16)] = jnp.ones((16,), jnp.float32)

                @pl.loop(0, nchunk_t)
                def _(k):
                    base = off + sid * epw_t + k * CHUNK
                    pltpu.sync_copy(dst.at[pl.ds(base, CHUNK)], didx)
                    pltpu.sync_copy(rows, acc.at[didx], add=True)
            else:
                @pl.loop(0, nchunk_t)
                def _(k):
                    base = sid * epw_t + k * CHUNK
                    pltpu.sync_copy(src.at[pl.ds(base, CHUNK)], sidx)
                    pltpu.sync_copy(dst.at[pl.ds(base, CHUNK)], didx)
                    pltpu.sync_copy(feat.at[sidx], rows)
                    pltpu.sync_copy(rows, acc.at[didx], add=True)
            plsc.subcore_barrier()

            sl = pl.ds(sid * ROWS_PER_SUB, ROWS_PER_SUB)
            pltpu.sync_copy(acc.at[sl], out.at[sl])
            plsc.subcore_barrier()


@jax.jit
def _sc_aggregate(au, bu, ai, bi, sf, df, sc_, dc, scb, dcb):
    mesh = plsc.VectorSubcoreMesh(core_axis_name="c", subcore_axis_name="s",
                                  num_cores=2, num_subcores=NSUB)
    sum_t = jax.ShapeDtypeStruct((NPAD, HALF), jnp.float32)
    cnt_t = jax.ShapeDtypeStruct((NPAD, HALF), jnp.float32)
    cp = pltpu.CompilerParams()
    if "needs_layout_passes" in pltpu.CompilerParams.__dataclass_fields__:
        cp = dataclasses.replace(cp, needs_layout_passes=False)
    kern = pl.kernel(
        _sc_body,
        out_type=[sum_t] * 6 + [cnt_t] * 4,
        mesh=mesh,
        compiler_params=cp,
        scratch_types=[
            pltpu.VMEM_SHARED((NPAD, HALF), jnp.float32),
            pltpu.VMEM((CHUNK, HALF), jnp.float32),
            pltpu.VMEM((CHUNK,), jnp.int32),
            pltpu.VMEM((CHUNK,), jnp.int32),
            pltpu.VMEM((ZROWS, HALF), jnp.float32),
        ],
    )
    return kern(au, bu, ai, bi, sf, df, sc_, dc, scb, dcb)


def _dot(x, w):
    return jax.lax.dot_general(
        x, w, (((1,), (0,)), ((), ())),
        precision=jax.lax.Precision.HIGHEST,
        preferred_element_type=jnp.float32,
    )


def _mean_linear(sa, sb, cnt, w_ref, b_ref):
    s = jnp.concatenate([sa, sb], axis=1)
    mean = s / jnp.maximum(cnt, 1.0)
    h = _dot(mean, w_ref[...])
    return h + b_ref[...] * (cnt > 0).astype(jnp.float32)


_BLK = 1000
_NB = N // _BLK


def _tc_body(sfa, sfb, cf, sba, sbb, cb, sca, scb_, cc0, cc1,
             wf, bf, wb, bb, wc, bc, out):
    i = pl.program_id(0)

    @pl.when(i < _NB)
    def _():
        hf = _mean_linear(sfa[...], sfb[...], cf[...][:, :1], wf, bf)
        hb = _mean_linear(sba[...], sbb[...], cb[...][:, :1], wb, bb)
        out[...] = hf + hb

    @pl.when(i >= _NB)
    def _():
        cnt = cc0[...][:, :1] + cc1[...][:, :1]
        out[...] = _mean_linear(sca[...], scb_[...], cnt, wc, bc)


def _row_spec():
    return pl.BlockSpec((_BLK, HALF), lambda i: (i % _NB, 0))


def _full_spec(shape):
    return pl.BlockSpec(shape, lambda i: (0, 0))


@jax.jit
def _tc_all(sfa, sfb, cf, sba, sbb, cb, sca, scb_, cc0, cc1,
            wf, bf, wb, bb, wc, bc):
    return pl.pallas_call(
        _tc_body,
        grid=(2 * _NB,),
        in_specs=[_row_spec()] * 10 +
                 [_full_spec((D, D)), _full_spec((1, D))] * 3,
        out_specs=pl.BlockSpec((_BLK, D), lambda i: (i, 0)),
        out_shape=jax.ShapeDtypeStruct((2 * N, D), jnp.float32),
    )(sfa, sfb, cf, sba, sbb, cb, sca, scb_, cc0, cc1,
      wf, bf, wb, bb, wc, bc)


def kernel(feat_user, feat_item, ei_follows, ei_clicks, ei_clicked_by,
           W_follows, b_follows, W_clicks, b_clicks, W_clicked_by, b_clicked_by):
    s_fa, s_fb, s_ca, s_cb, s_ba, s_bb, c_f, c_c0, c_c1, c_b = _sc_aggregate(
        feat_user[:, :HALF], feat_user[:, HALF:],
        feat_item[:, :HALF], feat_item[:, HALF:],
        ei_follows[0], ei_follows[1],
        ei_clicks[0], ei_clicks[1],
        ei_clicked_by[0], ei_clicked_by[1],
    )
    return _tc_all(s_fa, s_fb, c_f, s_ba, s_bb, c_b, s_ca, s_cb, c_c0, c_c1,
                   W_follows, b_follows.reshape(1, D),
                   W_clicked_by, b_clicked_by.reshape(1, D),
                   W_clicks, b_clicks.reshape(1, D))


# speedup vs baseline: 4.0476x; 1.0457x over previous
import dataclasses

import jax
import jax.numpy as jnp
from jax.experimental import pallas as pl
from jax.experimental.pallas import tpu as pltpu
from jax.experimental.pallas import tpu_sc as plsc

N = 10000
E = 160000
D = 256
HALF = 128
NSUB = 16
EPW = E // NSUB
CHUNK = 200
NCHUNK = EPW // CHUNK
NPAD = 10240
ROWS_PER_SUB = NPAD // NSUB
ZROWS = 32
EPWH = EPW // 2


def _sc_body(au, bu, ai, bi, ii_f, ii_c, ii_b, df, dc, dcb,
             o_fa, o_fb, o_ca, o_cb_, o_ba, o_bb, o_cf, o_cc0, o_cc1, o_cb2,
             acc, rows, idxb, didx, zbuf):
    cid = jax.lax.axis_index("c")
    sid = jax.lax.axis_index("s")

    @pl.loop(0, ZROWS)
    def _(r):
        @pl.loop(0, HALF, step=16)
        def _(c):
            zbuf[r, pl.ds(c, 16)] = jnp.zeros((16,), jnp.float32)

    @pl.loop(0, ROWS_PER_SUB, step=ZROWS)
    def _(r):
        pltpu.sync_copy(zbuf, acc.at[pl.ds(sid * ROWS_PER_SUB + r, ZROWS)])
    plsc.subcore_barrier()

    tasks = (
        (au, ii_f, o_fa, 0, False, False),
        (bu, ii_f, o_fb, 1, False, False),
        (au, ii_c, o_ca, 1, False, False),
        (bu, ii_c, o_cb_, 0, False, False),
        (ai, ii_b, o_ba, 0, False, False),
        (bi, ii_b, o_bb, 1, False, False),
        (None, df, o_cf, 0, False, True),
        (None, dc, o_cc0, 0, True, False),
        (None, dc, o_cc1, 1, True, True),
        (None, dcb, o_cb2, 1, False, False),
    )

    for feat, idx, out, core, half, fill1 in tasks:
        count = feat is None
        epw_t = EPWH if half else EPW
        nchunk_t = epw_t // CHUNK
        off = core * NSUB * EPWH if half else 0

        @pl.when(cid == core)
        def _():
            if count:
                if fill1:
                    @pl.loop(0, CHUNK)
                    def _(r):
                        @pl.loop(0, HALF, step=16)
                        def _(c):
                            rows[r, pl.ds(c, 16)] = jnp.ones((16,), jnp.float32)

                @pl.loop(0, nchunk_t)
                def _(k):
                    base = off + sid * epw_t + k * CHUNK
                    pltpu.sync_copy(idx.at[pl.ds(base, CHUNK)], didx)
                    pltpu.sync_copy(rows, acc.at[didx], add=True)
            else:
                @pl.loop(0, nchunk_t)
                def _(k):
                    base = (sid * nchunk_t + k) * 2 * CHUNK
                    pltpu.sync_copy(idx.at[pl.ds(base, 2 * CHUNK)], idxb)
                    pltpu.sync_copy(feat.at[idxb.at[pl.ds(0, CHUNK)]], rows)
                    pltpu.sync_copy(rows, acc.at[idxb.at[pl.ds(CHUNK, CHUNK)]],
                                    add=True)
            plsc.subcore_barrier()

            sl = pl.ds(sid * ROWS_PER_SUB, ROWS_PER_SUB)
            pltpu.sync_copy(acc.at[sl], out.at[sl])
            @pl.loop(0, ROWS_PER_SUB, step=ZROWS)
            def _(r):
                pltpu.sync_copy(zbuf, acc.at[pl.ds(sid * ROWS_PER_SUB + r, ZROWS)])
            plsc.subcore_barrier()


@jax.jit
def _sc_aggregate(au, bu, ai, bi, ii_f, ii_c, ii_b, df, dc, dcb):
    mesh = plsc.VectorSubcoreMesh(core_axis_name="c", subcore_axis_name="s",
                                  num_cores=2, num_subcores=NSUB)
    sum_t = jax.ShapeDtypeStruct((NPAD, HALF), jnp.float32)
    cnt_t = jax.ShapeDtypeStruct((NPAD, HALF), jnp.float32)
    cp = pltpu.CompilerParams()
    if "needs_layout_passes" in pltpu.CompilerParams.__dataclass_fields__:
        cp = dataclasses.replace(cp, needs_layout_passes=False)
    kern = pl.kernel(
        _sc_body,
        out_type=[sum_t] * 6 + [cnt_t] * 4,
        mesh=mesh,
        compiler_params=cp,
        scratch_types=[
            pltpu.VMEM_SHARED((NPAD, HALF), jnp.float32),
            pltpu.VMEM((CHUNK, HALF), jnp.float32),
            pltpu.VMEM((2 * CHUNK,), jnp.int32),
            pltpu.VMEM((CHUNK,), jnp.int32),
            pltpu.VMEM((ZROWS, HALF), jnp.float32),
        ],
    )
    return kern(au, bu, ai, bi, ii_f, ii_c, ii_b, df, dc, dcb)


def _dot(x, w):
    return jax.lax.dot_general(
        x, w, (((1,), (0,)), ((), ())),
        precision=jax.lax.Precision.HIGHEST,
        preferred_element_type=jnp.float32,
    )


def _mean_linear(sa, sb, cnt, w_ref, b_ref):
    s = jnp.concatenate([sa, sb], axis=1)
    mean = s / jnp.maximum(cnt, 1.0)
    h = _dot(mean, w_ref[...])
    return h + b_ref[...] * (cnt > 0).astype(jnp.float32)


_BLK = 1000
_NB = N // _BLK


def _tc_body(sfa, sfb, cf, sba, sbb, cb, sca, scb_, cc0, cc1,
             wf, bf, wb, bb, wc, bc, out):
    i = pl.program_id(0)

    @pl.when(i < _NB)
    def _():
        hf = _mean_linear(sfa[...], sfb[...], cf[...][:, :1], wf, bf)
        hb = _mean_linear(sba[...], sbb[...], cb[...][:, :1], wb, bb)
        out[...] = hf + hb

    @pl.when(i >= _NB)
    def _():
        cnt = cc0[...][:, :1] + cc1[...][:, :1]
        out[...] = _mean_linear(sca[...], scb_[...], cnt, wc, bc)


def _row_spec():
    return pl.BlockSpec((_BLK, HALF), lambda i: (i % _NB, 0))


def _full_spec(shape):
    return pl.BlockSpec(shape, lambda i: (0, 0))


@jax.jit
def _tc_all(sfa, sfb, cf, sba, sbb, cb, sca, scb_, cc0, cc1,
            wf, bf, wb, bb, wc, bc):
    return pl.pallas_call(
        _tc_body,
        grid=(2 * _NB,),
        in_specs=[_row_spec()] * 10 +
                 [_full_spec((D, D)), _full_spec((1, D))] * 3,
        out_specs=pl.BlockSpec((_BLK, D), lambda i: (i, 0)),
        out_shape=jax.ShapeDtypeStruct((2 * N, D), jnp.float32),
    )(sfa, sfb, cf, sba, sbb, cb, sca, scb_, cc0, cc1,
      wf, bf, wb, bb, wc, bc)


def _ileave(ei):
    return ei.reshape(2, NSUB, NCHUNK, CHUNK).transpose(1, 2, 0, 3).reshape(-1)


def kernel(feat_user, feat_item, ei_follows, ei_clicks, ei_clicked_by,
           W_follows, b_follows, W_clicks, b_clicks, W_clicked_by, b_clicked_by):
    s_fa, s_fb, s_ca, s_cb, s_ba, s_bb, c_f, c_c0, c_c1, c_b = _sc_aggregate(
        feat_user[:, :HALF], feat_user[:, HALF:],
        feat_item[:, :HALF], feat_item[:, HALF:],
        _ileave(ei_follows), _ileave(ei_clicks), _ileave(ei_clicked_by),
        ei_follows[1], ei_clicks[1], ei_clicked_by[1],
    )
    return _tc_all(s_fa, s_fb, c_f, s_ba, s_bb, c_b, s_ca, s_cb, c_c0, c_c1,
                   W_follows, b_follows.reshape(1, D),
                   W_clicked_by, b_clicked_by.reshape(1, D),
                   W_clicks, b_clicks.reshape(1, D))


# speedup vs baseline: 4.1151x; 1.0167x over previous
import dataclasses

import jax
import jax.numpy as jnp
from jax.experimental import pallas as pl
from jax.experimental.pallas import tpu as pltpu
from jax.experimental.pallas import tpu_sc as plsc

N = 10000
E = 160000
D = 256
HALF = 128
NSUB = 16
EPW = E // NSUB
CHUNK = 200
NCHUNK = EPW // CHUNK
NPAD = 10240
ROWS_PER_SUB = NPAD // NSUB
ZROWS = 32
EPWH = EPW // 2


def _sc_body(au, bu, ai, bi, ii_f, ii_c, ii_b, df, dc, dcb,
             o_fa, o_fb, o_ca, o_cb_, o_ba, o_bb, o_cf, o_cc0, o_cc1, o_cb2,
             acc, rows, idxb, didx, zbuf):
    cid = jax.lax.axis_index("c")
    sid = jax.lax.axis_index("s")

    @pl.loop(0, ZROWS)
    def _(r):
        @pl.loop(0, HALF, step=16)
        def _(c):
            zbuf[r, pl.ds(c, 16)] = jnp.zeros((16,), jnp.float32)

    @pl.loop(0, ROWS_PER_SUB, step=ZROWS)
    def _(r):
        pltpu.sync_copy(zbuf, acc.at[pl.ds(sid * ROWS_PER_SUB + r, ZROWS)])
    plsc.subcore_barrier()

    tasks = (
        (au, ii_f, o_fa, 0, False, False, False),
        (bu, ii_f, o_fb, 1, False, False, False),
        (au, ii_c, o_ca, 1, False, False, False),
        (bu, ii_c, o_cb_, 0, False, False, False),
        (ai, ii_b, o_ba, 0, False, False, False),
        (bi, ii_b, o_bb, 1, False, False, False),
        (None, df, o_cf, 0, False, True, False),
        (None, dc, o_cc0, 0, True, False, True),
        (None, dc, o_cc1, 1, True, True, False),
        (None, dcb, o_cb2, 1, False, False, True),
    )

    for feat, idx, out, core, half, fill1, last in tasks:
        count = feat is None
        pair = count and not half
        epw_t = EPWH if half else EPW
        nchunk_t = epw_t // (2 * CHUNK) if (count and pair) else epw_t // CHUNK
        off = core * NSUB * EPWH if half else 0

        @pl.when(cid == core)
        def _():
            if count:
                if fill1:
                    @pl.loop(0, CHUNK)
                    def _(r):
                        @pl.loop(0, HALF, step=16)
                        def _(c):
                            rows[r, pl.ds(c, 16)] = jnp.ones((16,), jnp.float32)

                if pair:
                    @pl.loop(0, nchunk_t)
                    def _(k):
                        base = off + sid * epw_t + k * 2 * CHUNK
                        pltpu.sync_copy(idx.at[pl.ds(base, 2 * CHUNK)], idxb)
                        pltpu.sync_copy(rows, acc.at[idxb.at[pl.ds(0, CHUNK)]],
                                        add=True)
                        pltpu.sync_copy(rows,
                                        acc.at[idxb.at[pl.ds(CHUNK, CHUNK)]],
                                        add=True)
                else:
                    @pl.loop(0, nchunk_t)
                    def _(k):
                        base = off + sid * epw_t + k * CHUNK
                        pltpu.sync_copy(idx.at[pl.ds(base, CHUNK)], didx)
                        pltpu.sync_copy(rows, acc.at[didx], add=True)
            else:
                @pl.loop(0, nchunk_t)
                def _(k):
                    base = (sid * nchunk_t + k) * 2 * CHUNK
                    pltpu.sync_copy(idx.at[pl.ds(base, 2 * CHUNK)], idxb)
                    pltpu.sync_copy(feat.at[idxb.at[pl.ds(0, CHUNK)]], rows)
                    pltpu.sync_copy(rows, acc.at[idxb.at[pl.ds(CHUNK, CHUNK)]],
                                    add=True)
            plsc.subcore_barrier()

            sl = pl.ds(sid * ROWS_PER_SUB, ROWS_PER_SUB)
            pltpu.sync_copy(acc.at[sl], out.at[sl])
            if not last:
                @pl.loop(0, ROWS_PER_SUB, step=ZROWS)
                def _(r):
                    pltpu.sync_copy(zbuf, acc.at[pl.ds(sid * ROWS_PER_SUB + r, ZROWS)])
            plsc.subcore_barrier()


@jax.jit
def _sc_aggregate(au, bu, ai, bi, ii_f, ii_c, ii_b, df, dc, dcb):
    mesh = plsc.VectorSubcoreMesh(core_axis_name="c", subcore_axis_name="s",
                                  num_cores=2, num_subcores=NSUB)
    sum_t = jax.ShapeDtypeStruct((NPAD, HALF), jnp.float32)
    cnt_t = jax.ShapeDtypeStruct((NPAD, HALF), jnp.float32)
    cp = pltpu.CompilerParams()
    if "needs_layout_passes" in pltpu.CompilerParams.__dataclass_fields__:
        cp = dataclasses.replace(cp, needs_layout_passes=False)
    kern = pl.kernel(
        _sc_body,
        out_type=[sum_t] * 6 + [cnt_t] * 4,
        mesh=mesh,
        compiler_params=cp,
        scratch_types=[
            pltpu.VMEM_SHARED((NPAD, HALF), jnp.float32),
            pltpu.VMEM((CHUNK, HALF), jnp.float32),
            pltpu.VMEM((2 * CHUNK,), jnp.int32),
            pltpu.VMEM((CHUNK,), jnp.int32),
            pltpu.VMEM((ZROWS, HALF), jnp.float32),
        ],
    )
    return kern(au, bu, ai, bi, ii_f, ii_c, ii_b, df, dc, dcb)


def _dot(x, w):
    return jax.lax.dot_general(
        x, w, (((1,), (0,)), ((), ())),
        precision=jax.lax.Precision.HIGHEST,
        preferred_element_type=jnp.float32,
    )


def _mean_linear(sa, sb, cnt, w_ref, b_ref):
    s = jnp.concatenate([sa, sb], axis=1)
    mean = s / jnp.maximum(cnt, 1.0)
    h = _dot(mean, w_ref[...])
    return h + b_ref[...] * (cnt > 0).astype(jnp.float32)


_BLK = 1000
_NB = N // _BLK


def _tc_body(sfa, sfb, cf, sba, sbb, cb, sca, scb_, cc0, cc1,
             wf, bf, wb, bb, wc, bc, out):
    i = pl.program_id(0)

    @pl.when(i < _NB)
    def _():
        hf = _mean_linear(sfa[...], sfb[...], cf[...][:, :1], wf, bf)
        hb = _mean_linear(sba[...], sbb[...], cb[...][:, :1], wb, bb)
        out[...] = hf + hb

    @pl.when(i >= _NB)
    def _():
        cnt = cc0[...][:, :1] + cc1[...][:, :1]
        out[...] = _mean_linear(sca[...], scb_[...], cnt, wc, bc)


def _row_spec():
    return pl.BlockSpec((_BLK, HALF), lambda i: (i % _NB, 0))


def _full_spec(shape):
    return pl.BlockSpec(shape, lambda i: (0, 0))


@jax.jit
def _tc_all(sfa, sfb, cf, sba, sbb, cb, sca, scb_, cc0, cc1,
            wf, bf, wb, bb, wc, bc):
    return pl.pallas_call(
        _tc_body,
        grid=(2 * _NB,),
        in_specs=[_row_spec()] * 10 +
                 [_full_spec((D, D)), _full_spec((1, D))] * 3,
        out_specs=pl.BlockSpec((_BLK, D), lambda i: (i, 0)),
        out_shape=jax.ShapeDtypeStruct((2 * N, D), jnp.float32),
    )(sfa, sfb, cf, sba, sbb, cb, sca, scb_, cc0, cc1,
      wf, bf, wb, bb, wc, bc)


def _ileave(ei):
    return ei.reshape(2, NSUB, NCHUNK, CHUNK).transpose(1, 2, 0, 3).reshape(-1)


def kernel(feat_user, feat_item, ei_follows, ei_clicks, ei_clicked_by,
           W_follows, b_follows, W_clicks, b_clicks, W_clicked_by, b_clicked_by):
    s_fa, s_fb, s_ca, s_cb, s_ba, s_bb, c_f, c_c0, c_c1, c_b = _sc_aggregate(
        feat_user[:, :HALF], feat_user[:, HALF:],
        feat_item[:, :HALF], feat_item[:, HALF:],
        _ileave(ei_follows), _ileave(ei_clicks), _ileave(ei_clicked_by),
        ei_follows[1], ei_clicks[1], ei_clicked_by[1],
    )
    return _tc_all(s_fa, s_fb, c_f, s_ba, s_bb, c_b, s_ca, s_cb, c_c0, c_c1,
                   W_follows, b_follows.reshape(1, D),
                   W_clicked_by, b_clicked_by.reshape(1, D),
                   W_clicks, b_clicks.reshape(1, D))
